# Initial kernel scaffold; baseline (speedup 1.0000x reference)
import functools

import jax
import jax.numpy as jnp
from jax import lax
from jax.experimental import pallas as pl
from jax.experimental.pallas import tpu as pltpu
from jax.experimental.pallas import tpu_sc as plsc

F32 = jnp.float32

_N = 50000
_E = 800000
_H = 64
_NC = 2
_NS = 16
_NW = _NC * _NS
_EPW = _E // _NW
_NPAD = 50048
_RPT = _NPAD // _NS
_C = 1000
_NCH = _EPW // _C

_BE = 4000
_NBE = _E // _BE
_BN = 2000
_NBN = _N // _BN


def _sc_mesh():
    return plsc.VectorSubcoreMesh(core_axis_name="c", subcore_axis_name="s",
                                  num_cores=_NC, num_subcores=_NS)


def _sc_gather_x(x, srcE, dstE, onesC, zcnt):
    @functools.partial(
        pl.kernel,
        out_type=(
            jax.ShapeDtypeStruct((_E, 1), F32),
            jax.ShapeDtypeStruct((_E, 1), F32),
            jax.ShapeDtypeStruct((_NC, _NPAD), F32),
        ),
        mesh=_sc_mesh(),
        scratch_types=[
            pltpu.VMEM((_C,), jnp.int32),
            pltpu.VMEM((_C, 1), F32),
            pltpu.VMEM((_C,), F32),
            pltpu.VMEM_SHARED((_NPAD,), F32),
            pltpu.SemaphoreType.DMA,
        ],
    )
    def k(x_h, src_h, dst_h, ones_h, zc_h, xi_h, xj_h, cnt_h,
          idx_v, val_v, ones_v, cnt_sh, sem):
        cid = lax.axis_index("c")
        sid = lax.axis_index("s")
        wid = sid * _NC + cid
        base = wid * _EPW
        pltpu.sync_copy(zc_h, cnt_sh.at[pl.ds(sid * _RPT, _RPT)])
        pltpu.sync_copy(ones_h, ones_v)
        plsc.subcore_barrier()

        def chunk(kk, carry):
            e0 = base + kk * _C
            pltpu.sync_copy(dst_h.at[pl.ds(e0, _C)], idx_v)
            pltpu.async_copy(x_h.at[idx_v], val_v, sem).wait()
            pltpu.sync_copy(val_v, xi_h.at[pl.ds(e0, _C)])
            pltpu.sync_copy(ones_v, cnt_sh.at[idx_v], add=True)
            pltpu.sync_copy(src_h.at[pl.ds(e0, _C)], idx_v)
            pltpu.async_copy(x_h.at[idx_v], val_v, sem).wait()
            pltpu.sync_copy(val_v, xj_h.at[pl.ds(e0, _C)])
            return carry

        lax.fori_loop(0, _NCH, chunk, 0)
        plsc.subcore_barrier()
        pltpu.sync_copy(cnt_sh.at[pl.ds(sid * _RPT, _RPT)],
                        cnt_h.at[cid, pl.ds(sid * _RPT, _RPT)])

    return k(x, srcE, dstE, onesC, zcnt)


def _sc_scatter(mL, mR, dstE, z32):
    @functools.partial(
        pl.kernel,
        out_type=jax.ShapeDtypeStruct((_NC, _NPAD, _H), F32),
        mesh=_sc_mesh(),
        scratch_types=[
            pltpu.VMEM((_C,), jnp.int32),
            pltpu.VMEM((_C, 32), F32),
            pltpu.VMEM_SHARED((_NPAD, 32), F32),
        ],
    )
    def k(mL_h, mR_h, dst_h, z_h, sum_h, idx_v, data_v, tab_sh):
        cid = lax.axis_index("c")
        sid = lax.axis_index("s")
        wid = sid * _NC + cid
        base = wid * _EPW
        for half in range(2):
            m_h = (mL_h, mR_h)[half]
            pltpu.sync_copy(z_h, tab_sh.at[pl.ds(sid * _RPT, _RPT), :])
            plsc.subcore_barrier()

            def chunk(kk, carry):
                e0 = base + kk * _C
                pltpu.sync_copy(dst_h.at[pl.ds(e0, _C)], idx_v)
                pltpu.sync_copy(m_h.at[pl.ds(e0, _C), :], data_v)
                pltpu.sync_copy(data_v, tab_sh.at[idx_v], add=True)
                return carry

            lax.fori_loop(0, _NCH, chunk, 0)
            plsc.subcore_barrier()
            pltpu.sync_copy(tab_sh.at[pl.ds(sid * _RPT, _RPT), :],
                            sum_h.at[cid, pl.ds(sid * _RPT, _RPT),
                                     pl.ds(32 * half, 32)])
            plsc.subcore_barrier()

    return k(mL, mR, dstE, z32)


def _sc_gather_rows(A, B, srcE, dstE):
    @functools.partial(
        pl.kernel,
        out_type=(
            jax.ShapeDtypeStruct((_E, _H), F32),
            jax.ShapeDtypeStruct((_E, _H), F32),
        ),
        mesh=_sc_mesh(),
        scratch_types=[
            pltpu.VMEM((_C,), jnp.int32),
            pltpu.VMEM((_C, _H), F32),
            pltpu.SemaphoreType.DMA,
        ],
    )
    def k(a_h, b_h, src_h, dst_h, ga_h, gb_h, idx_v, buf_v, sem):
        cid = lax.axis_index("c")
        sid = lax.axis_index("s")
        wid = sid * _NC + cid
        base = wid * _EPW

        def chunk(kk, carry):
            e0 = base + kk * _C
            pltpu.sync_copy(dst_h.at[pl.ds(e0, _C)], idx_v)
            pltpu.async_copy(a_h.at[idx_v], buf_v, sem).wait()
            pltpu.sync_copy(buf_v, ga_h.at[pl.ds(e0, _C), :])
            pltpu.sync_copy(src_h.at[pl.ds(e0, _C)], idx_v)
            pltpu.async_copy(b_h.at[idx_v], buf_v, sem).wait()
            pltpu.sync_copy(buf_v, gb_h.at[pl.ds(e0, _C), :])
            return carry

        lax.fori_loop(0, _NCH, chunk, 0)

    return k(A, B, srcE, dstE)


def _full(shape):
    return pl.BlockSpec(shape, lambda i: tuple(0 for _ in shape))


def _tc_msg1(xi, xj, ea, W1, b1, W2, b2):
    def body(xi_r, xj_r, ea_r, W1_r, b1_r, W2_r, b2_r, mL_r, mR_r):
        pre = (xi_r[...] * W1_r[0:1, :] + xj_r[...] * W1_r[1:2, :]
               + ea_r[...] * W1_r[2:3, :] + b1_r[...])
        m = jnp.maximum(pre, 0.0)
        out = jnp.dot(m, W2_r[...], preferred_element_type=F32) + b2_r[...]
        mL_r[...] = out[:, :32]
        mR_r[...] = out[:, 32:]

    eb = pl.BlockSpec((_BE, 1), lambda i: (i, 0))
    return pl.pallas_call(
        body,
        grid=(_NBE,),
        in_specs=[eb, eb, eb, _full((3, _H)), _full((_H,)),
                  _full((_H, _H)), _full((_H,))],
        out_specs=[pl.BlockSpec((_BE, 32), lambda i: (i, 0))] * 2,
        out_shape=[jax.ShapeDtypeStruct((_E, 32), F32)] * 2,
    )(xi, xj, ea, W1, b1, W2, b2)


def _tc_update1(x, sumP, cntP, uWx, uWa, ub, Wd, bm, Ws):
    def body(x_r, S_r, c_r, uWx_r, uWa_r, ub_r, Wd_r, bm_r, Ws_r,
             h1_r, A_r, B_r):
        s = S_r[0] + S_r[1]
        cnt = jnp.maximum(c_r[0] + c_r[1], 1.0)
        agg = s / cnt
        u = (x_r[...] * uWx_r[...]
             + jnp.dot(agg, uWa_r[...], preferred_element_type=F32) + ub_r[...])
        h = jnp.maximum(u, 0.0)
        h1_r[...] = h
        A_r[...] = jnp.dot(h, Wd_r[...], preferred_element_type=F32) + bm_r[...]
        B_r[...] = jnp.dot(h, Ws_r[...], preferred_element_type=F32)

    return pl.pallas_call(
        body,
        grid=(_NBN,),
        in_specs=[
            pl.BlockSpec((_BN, 1), lambda i: (i, 0)),
            pl.BlockSpec((2, _BN, _H), lambda i: (0, i, 0)),
            pl.BlockSpec((2, _BN, 1), lambda i: (0, i, 0)),
            _full((1, _H)), _full((_H, _H)), _full((_H,)),
            _full((_H, _H)), _full((_H,)), _full((_H, _H)),
        ],
        out_specs=[pl.BlockSpec((_BN, _H), lambda i: (i, 0))] * 3,
        out_shape=[jax.ShapeDtypeStruct((_N, _H), F32)] * 3,
    )(x, sumP, cntP, uWx, uWa, ub, Wd, bm, Ws)


def _tc_msg2(GA, GB, ea, we, W2, b2):
    def body(ga_r, gb_r, ea_r, we_r, W2_r, b2_r, mL_r, mR_r):
        pre = ga_r[...] + gb_r[...] + ea_r[...] * we_r[...]
        m = jnp.maximum(pre, 0.0)
        out = jnp.dot(m, W2_r[...], preferred_element_type=F32) + b2_r[...]
        mL_r[...] = out[:, :32]
        mR_r[...] = out[:, 32:]

    ebH = pl.BlockSpec((_BE, _H), lambda i: (i, 0))
    return pl.pallas_call(
        body,
        grid=(_NBE,),
        in_specs=[ebH, ebH, pl.BlockSpec((_BE, 1), lambda i: (i, 0)),
                  _full((1, _H)), _full((_H, _H)), _full((_H,))],
        out_specs=[pl.BlockSpec((_BE, 32), lambda i: (i, 0))] * 2,
        out_shape=[jax.ShapeDtypeStruct((_E, 32), F32)] * 2,
    )(GA, GB, ea, we, W2, b2)


def _tc_update2(h1, sumP, cntP, scal, uWh, uWa, ub, fc1Wp, fc1s, fc1b,
                fc2W, fc2b):
    def body(h1_r, S_r, c_r, scal_r, uWh_r, uWa_r, ub_r, W1p_r, W1s_r, b1_r,
             W2_r, b2_r, out_r, acc):
        i = pl.program_id(0)
        s = S_r[0] + S_r[1]
        cnt = jnp.maximum(c_r[0] + c_r[1], 1.0)
        agg = s / cnt
        u = (jnp.dot(h1_r[...], uWh_r[...], preferred_element_type=F32)
             + jnp.dot(agg, uWa_r[...], preferred_element_type=F32)
             + ub_r[...])
        h2 = jnp.maximum(u, 0.0)
        ps = jnp.sum(h2, axis=0, keepdims=True)

        @pl.when(i == 0)
        def _init():
            acc[0:1, 0:_H] = ps

        @pl.when(i > 0)
        def _accum():
            acc[0:1, 0:_H] = acc[0:1, 0:_H] + ps

        @pl.when(i == _NBN - 1)
        def _finish():
            pooled = acc[0:1, 0:_H] / jnp.float32(_N)
            u1 = (jnp.dot(pooled, W1p_r[...], preferred_element_type=F32)
                  + scal_r[0] * W1s_r[0:1, :] + scal_r[1] * W1s_r[1:2, :]
                  + b1_r[...])
            r1 = jnp.maximum(u1, 0.0)
            z = jnp.dot(r1, W2_r[...], preferred_element_type=F32) + b2_r[...]
            out_r[...] = z

    return pl.pallas_call(
        body,
        grid=(_NBN,),
        in_specs=[
            pl.BlockSpec((_BN, _H), lambda i: (i, 0)),
            pl.BlockSpec((2, _BN, _H), lambda i: (0, i, 0)),
            pl.BlockSpec((2, _BN, 1), lambda i: (0, i, 0)),
            pl.BlockSpec(memory_space=pltpu.SMEM),
            _full((_H, _H)), _full((_H, _H)), _full((_H,)),
            _full((_H, _H)), _full((2, _H)), _full((_H,)),
            _full((_H, 1)), _full((1,)),
        ],
        out_specs=pl.BlockSpec((1, 1), lambda i: (0, 0)),
        out_shape=jax.ShapeDtypeStruct((1, 1), F32),
        scratch_shapes=[pltpu.VMEM((8, 128), F32)],
    )(h1, sumP, cntP, scal, uWh, uWa, ub, fc1Wp, fc1s, fc1b, fc2W, fc2b)


def kernel(x, edge_index, edge_attr, theta, log_h, batch,
           c1m1W, c1m1b, c1m2W, c1m2b, c1uW, c1ub,
           c2m1W, c2m1b, c2m2W, c2m2b, c2uW, c2ub,
           fc1W, fc1b, fc2W, fc2b):
    src = edge_index[0]
    dst = edge_index[1]
    onesC = jnp.ones((_C,), F32)
    zcnt = jnp.zeros((_RPT,), F32)
    z32 = jnp.zeros((_RPT, 32), F32)

    xi, xj, cntP = _sc_gather_x(x, src, dst, onesC, zcnt)
    mL1, mR1 = _tc_msg1(xi, xj, edge_attr, c1m1W, c1m1b, c1m2W, c1m2b)
    sum1 = _sc_scatter(mL1, mR1, dst, z32)
    cnt3 = cntP.reshape((_NC, _NPAD, 1))
    h1, A2, B2 = _tc_update1(x, sum1, cnt3, c1uW[0:1], c1uW[1:], c1ub,
                             c2m1W[0:_H], c2m1b, c2m1W[_H:2 * _H])
    GA, GB = _sc_gather_rows(A2, B2, src, dst)
    mL2, mR2 = _tc_msg2(GA, GB, edge_attr, c2m1W[2 * _H:2 * _H + 1],
                        c2m2W, c2m2b)
    sum2 = _sc_scatter(mL2, mR2, dst, z32)
    scal = jnp.concatenate([theta, log_h]).astype(F32)
    out = _tc_update2(h1, sum2, cnt3, scal, c2uW[0:_H], c2uW[_H:], c2ub,
                      fc1W[0:_H], fc1W[_H:_H + 2], fc1b, fc2W, fc2b)
    return out.reshape((1,))


# speedup vs baseline: 2.7301x; 2.7301x over previous
import functools

import jax
import jax.numpy as jnp
from jax import lax
from jax.experimental import pallas as pl
from jax.experimental.pallas import tpu as pltpu
from jax.experimental.pallas import tpu_sc as plsc

F32 = jnp.float32

_N = 50000
_E = 800000
_H = 64
_NC = 2
_NS = 16
_NW = _NC * _NS
_EPW = _E // _NW
_NPAD = 51200
_RPT = _NPAD // _NS
_PZ = 800
_NPZ = _RPT // _PZ
_QW = 16
_C = 1000
_NCH = _EPW // _C

_BE = 4000
_NBE = _E // _BE
_BN = 2000
_NBN = _N // _BN


def _sc_mesh():
    return plsc.VectorSubcoreMesh(core_axis_name="c", subcore_axis_name="s",
                                  num_cores=_NC, num_subcores=_NS)


_SC_PARAMS = pltpu.CompilerParams(use_tc_tiling_on_sc=False)


def _sc_gather_x(x, srcE, dstE, onesC, zcnt):
    @functools.partial(
        pl.kernel,
        out_type=(
            jax.ShapeDtypeStruct((_E, 1), F32),
            jax.ShapeDtypeStruct((_E, 1), F32),
            jax.ShapeDtypeStruct((_NC * _NPAD,), F32),
        ),
        mesh=_sc_mesh(),
        compiler_params=_SC_PARAMS,
        scratch_types=[
            pltpu.VMEM((_C,), jnp.int32),
            pltpu.VMEM((_C, 1), F32),
            pltpu.VMEM((_C,), F32),
            pltpu.VMEM((_RPT,), F32),
            pltpu.VMEM_SHARED((_NPAD,), F32),
            pltpu.SemaphoreType.DMA,
        ],
    )
    def k(x_h, src_h, dst_h, ones_h, zc_h, xi_h, xj_h, cnt_h,
          idx_v, val_v, ones_v, cbuf_v, cnt_sh, sem):
        cid = lax.axis_index("c")
        sid = lax.axis_index("s")
        wid = sid * _NC + cid
        base = wid * _EPW
        pltpu.sync_copy(zc_h, cbuf_v)
        pltpu.sync_copy(cbuf_v, cnt_sh.at[pl.ds(sid * _RPT, _RPT)])
        pltpu.sync_copy(ones_h, ones_v)
        plsc.subcore_barrier()

        def chunk(kk, carry):
            e0 = base + kk * _C
            pltpu.sync_copy(dst_h.at[pl.ds(e0, _C)], idx_v)
            pltpu.async_copy(x_h.at[idx_v], val_v, sem).wait()
            pltpu.sync_copy(val_v, xi_h.at[pl.ds(e0, _C)])
            pltpu.sync_copy(ones_v, cnt_sh.at[idx_v], add=True)
            pltpu.sync_copy(src_h.at[pl.ds(e0, _C)], idx_v)
            pltpu.async_copy(x_h.at[idx_v], val_v, sem).wait()
            pltpu.sync_copy(val_v, xj_h.at[pl.ds(e0, _C)])
            return carry

        lax.fori_loop(0, _NCH, chunk, 0)
        plsc.subcore_barrier()
        pltpu.sync_copy(cnt_sh.at[pl.ds(sid * _RPT, _RPT)], cbuf_v)
        pltpu.sync_copy(cbuf_v, cnt_h.at[pl.ds(cid * _NPAD + sid * _RPT, _RPT)])

    return k(x, srcE, dstE, onesC, zcnt)


def _sc_scatter(mQ, dstE, zq):
    @functools.partial(
        pl.kernel,
        out_type=jax.ShapeDtypeStruct((_NC, 4, _NPAD, _QW), F32),
        mesh=_sc_mesh(),
        compiler_params=_SC_PARAMS,
        scratch_types=[
            pltpu.VMEM((_C,), jnp.int32),
            pltpu.VMEM((_C, _QW), F32),
            pltpu.VMEM((_PZ, _QW), F32),
            pltpu.VMEM_SHARED((_NPAD, _QW), F32),
        ],
    )
    def k(m0_h, m1_h, m2_h, m3_h, dst_h, z_h, sum_h, idx_v, data_v, zb_v,
          tab_sh):
        cid = lax.axis_index("c")
        sid = lax.axis_index("s")
        wid = sid * _NC + cid
        base = wid * _EPW
        pltpu.sync_copy(z_h, zb_v)
        for q in range(4):
            m_h = (m0_h, m1_h, m2_h, m3_h)[q]
            for p in range(_NPZ):
                pltpu.sync_copy(
                    zb_v, tab_sh.at[pl.ds(sid * _RPT + p * _PZ, _PZ), :])
            plsc.subcore_barrier()

            def chunk(kk, carry):
                e0 = base + kk * _C
                pltpu.sync_copy(dst_h.at[pl.ds(e0, _C)], idx_v)
                pltpu.sync_copy(m_h.at[pl.ds(e0, _C), :], data_v)
                pltpu.sync_copy(data_v, tab_sh.at[idx_v], add=True)
                return carry

            lax.fori_loop(0, _NCH, chunk, 0)
            plsc.subcore_barrier()
            for p in range(_NPZ):
                r0 = sid * _RPT + p * _PZ
                pltpu.sync_copy(tab_sh.at[pl.ds(r0, _PZ), :], zb_v)
                pltpu.sync_copy(zb_v, sum_h.at[cid, q, pl.ds(r0, _PZ), :])
            pltpu.sync_copy(z_h, zb_v)
            plsc.subcore_barrier()

    return k(*mQ, dstE, zq)


def _sc_gather_rows(A, B, srcE, dstE):
    @functools.partial(
        pl.kernel,
        out_type=(
            jax.ShapeDtypeStruct((_E, _H), F32),
            jax.ShapeDtypeStruct((_E, _H), F32),
        ),
        mesh=_sc_mesh(),
        compiler_params=_SC_PARAMS,
        scratch_types=[
            pltpu.VMEM((_C,), jnp.int32),
            pltpu.VMEM((_C, _H), F32),
            pltpu.SemaphoreType.DMA,
        ],
    )
    def k(a_h, b_h, src_h, dst_h, ga_h, gb_h, idx_v, buf_v, sem):
        cid = lax.axis_index("c")
        sid = lax.axis_index("s")
        wid = sid * _NC + cid
        base = wid * _EPW

        def chunk(kk, carry):
            e0 = base + kk * _C
            pltpu.sync_copy(dst_h.at[pl.ds(e0, _C)], idx_v)
            pltpu.async_copy(a_h.at[idx_v], buf_v, sem).wait()
            pltpu.sync_copy(buf_v, ga_h.at[pl.ds(e0, _C), :])
            pltpu.sync_copy(src_h.at[pl.ds(e0, _C)], idx_v)
            pltpu.async_copy(b_h.at[idx_v], buf_v, sem).wait()
            pltpu.sync_copy(buf_v, gb_h.at[pl.ds(e0, _C), :])
            return carry

        lax.fori_loop(0, _NCH, chunk, 0)

    return k(A, B, srcE, dstE)


def _full(shape):
    return pl.BlockSpec(shape, lambda i: tuple(0 for _ in shape))


def _tc_msg1(xi, xj, ea, W1, b1, W2, b2):
    def body(xi_r, xj_r, ea_r, W1_r, b1_r, W2_r, b2_r, m0_r, m1_r, m2_r, m3_r):
        pre = (xi_r[...] * W1_r[0:1, :] + xj_r[...] * W1_r[1:2, :]
               + ea_r[...] * W1_r[2:3, :] + b1_r[...])
        m = jnp.maximum(pre, 0.0)
        out = jnp.dot(m, W2_r[...], preferred_element_type=F32) + b2_r[...]
        for q, mq_r in enumerate((m0_r, m1_r, m2_r, m3_r)):
            mq_r[...] = out[:, q * _QW:(q + 1) * _QW]

    eb = pl.BlockSpec((_BE, 1), lambda i: (i, 0))
    return pl.pallas_call(
        body,
        grid=(_NBE,),
        in_specs=[eb, eb, eb, _full((3, _H)), _full((_H,)),
                  _full((_H, _H)), _full((_H,))],
        out_specs=[pl.BlockSpec((_BE, _QW), lambda i: (i, 0))] * 4,
        out_shape=[jax.ShapeDtypeStruct((_E, _QW), F32)] * 4,
    )(xi, xj, ea, W1, b1, W2, b2)


def _tc_update1(x, sumP, cntP, uWx, uWa, ub, Wd, bm, Ws):
    def body(x_r, S_r, c_r, uWx_r, uWa_r, ub_r, Wd_r, bm_r, Ws_r,
             h1_r, A_r, B_r):
        s = jnp.concatenate([S_r[0, q] + S_r[1, q] for q in range(4)], axis=-1)
        cnt = jnp.maximum(c_r[0] + c_r[1], 1.0)
        agg = s / cnt
        u = (x_r[...] * uWx_r[...]
             + jnp.dot(agg, uWa_r[...], preferred_element_type=F32) + ub_r[...])
        h = jnp.maximum(u, 0.0)
        h1_r[...] = h
        A_r[...] = jnp.dot(h, Wd_r[...], preferred_element_type=F32) + bm_r[...]
        B_r[...] = jnp.dot(h, Ws_r[...], preferred_element_type=F32)

    return pl.pallas_call(
        body,
        grid=(_NBN,),
        in_specs=[
            pl.BlockSpec((_BN, 1), lambda i: (i, 0)),
            pl.BlockSpec((2, 4, _BN, _QW), lambda i: (0, 0, i, 0)),
            pl.BlockSpec((2, _BN, 1), lambda i: (0, i, 0)),
            _full((1, _H)), _full((_H, _H)), _full((_H,)),
            _full((_H, _H)), _full((_H,)), _full((_H, _H)),
        ],
        out_specs=[pl.BlockSpec((_BN, _H), lambda i: (i, 0))] * 3,
        out_shape=[jax.ShapeDtypeStruct((_N, _H), F32)] * 3,
    )(x, sumP, cntP, uWx, uWa, ub, Wd, bm, Ws)


def _tc_msg2(GA, GB, ea, we, W2, b2):
    def body(ga_r, gb_r, ea_r, we_r, W2_r, b2_r, m0_r, m1_r, m2_r, m3_r):
        pre = ga_r[...] + gb_r[...] + ea_r[...] * we_r[...]
        m = jnp.maximum(pre, 0.0)
        out = jnp.dot(m, W2_r[...], preferred_element_type=F32) + b2_r[...]
        for q, mq_r in enumerate((m0_r, m1_r, m2_r, m3_r)):
            mq_r[...] = out[:, q * _QW:(q + 1) * _QW]

    ebH = pl.BlockSpec((_BE, _H), lambda i: (i, 0))
    return pl.pallas_call(
        body,
        grid=(_NBE,),
        in_specs=[ebH, ebH, pl.BlockSpec((_BE, 1), lambda i: (i, 0)),
                  _full((1, _H)), _full((_H, _H)), _full((_H,))],
        out_specs=[pl.BlockSpec((_BE, _QW), lambda i: (i, 0))] * 4,
        out_shape=[jax.ShapeDtypeStruct((_E, _QW), F32)] * 4,
    )(GA, GB, ea, we, W2, b2)


def _tc_update2(h1, sumP, cntP, scal, uWh, uWa, ub, fc1Wp, fc1s, fc1b,
                fc2W, fc2b):
    def body(h1_r, S_r, c_r, scal_r, uWh_r, uWa_r, ub_r, W1p_r, W1s_r, b1_r,
             W2_r, b2_r, out_r, acc):
        i = pl.program_id(0)
        s = jnp.concatenate([S_r[0, q] + S_r[1, q] for q in range(4)], axis=-1)
        cnt = jnp.maximum(c_r[0] + c_r[1], 1.0)
        agg = s / cnt
        u = (jnp.dot(h1_r[...], uWh_r[...], preferred_element_type=F32)
             + jnp.dot(agg, uWa_r[...], preferred_element_type=F32)
             + ub_r[...])
        h2 = jnp.maximum(u, 0.0)
        ps = jnp.sum(h2, axis=0, keepdims=True)

        @pl.when(i == 0)
        def _init():
            acc[0:1, 0:_H] = ps

        @pl.when(i > 0)
        def _accum():
            acc[0:1, 0:_H] = acc[0:1, 0:_H] + ps

        @pl.when(i == _NBN - 1)
        def _finish():
            pooled = acc[0:1, 0:_H] / jnp.float32(_N)
            u1 = (jnp.dot(pooled, W1p_r[...], preferred_element_type=F32)
                  + scal_r[0] * W1s_r[0:1, :] + scal_r[1] * W1s_r[1:2, :]
                  + b1_r[...])
            r1 = jnp.maximum(u1, 0.0)
            z = jnp.dot(r1, W2_r[...], preferred_element_type=F32) + b2_r[...]
            out_r[...] = z

    return pl.pallas_call(
        body,
        grid=(_NBN,),
        in_specs=[
            pl.BlockSpec((_BN, _H), lambda i: (i, 0)),
            pl.BlockSpec((2, 4, _BN, _QW), lambda i: (0, 0, i, 0)),
            pl.BlockSpec((2, _BN, 1), lambda i: (0, i, 0)),
            pl.BlockSpec(memory_space=pltpu.SMEM),
            _full((_H, _H)), _full((_H, _H)), _full((_H,)),
            _full((_H, _H)), _full((2, _H)), _full((_H,)),
            _full((_H, 1)), _full((1,)),
        ],
        out_specs=pl.BlockSpec((1, 1), lambda i: (0, 0)),
        out_shape=jax.ShapeDtypeStruct((1, 1), F32),
        scratch_shapes=[pltpu.VMEM((8, 128), F32)],
    )(h1, sumP, cntP, scal, uWh, uWa, ub, fc1Wp, fc1s, fc1b, fc2W, fc2b)


def kernel(x, edge_index, edge_attr, theta, log_h, batch,
           c1m1W, c1m1b, c1m2W, c1m2b, c1uW, c1ub,
           c2m1W, c2m1b, c2m2W, c2m2b, c2uW, c2ub,
           fc1W, fc1b, fc2W, fc2b):
    src = edge_index[0]
    dst = edge_index[1]
    onesC = jnp.ones((_C,), F32)
    zcnt = jnp.zeros((_RPT,), F32)
    zq = jnp.zeros((_PZ, _QW), F32)

    xi, xj, cntP = _sc_gather_x(x, src, dst, onesC, zcnt)
    mQ1 = _tc_msg1(xi, xj, edge_attr, c1m1W, c1m1b, c1m2W, c1m2b)
    sum1 = _sc_scatter(mQ1, dst, zq)
    cnt3 = cntP.reshape((_NC, _NPAD, 1))
    h1, A2, B2 = _tc_update1(x, sum1, cnt3, c1uW[0:1], c1uW[1:], c1ub,
                             c2m1W[0:_H], c2m1b, c2m1W[_H:2 * _H])
    GA, GB = _sc_gather_rows(A2, B2, src, dst)
    mQ2 = _tc_msg2(GA, GB, edge_attr, c2m1W[2 * _H:2 * _H + 1],
                   c2m2W, c2m2b)
    sum2 = _sc_scatter(mQ2, dst, zq)
    scal = jnp.concatenate([theta, log_h]).astype(F32)
    out = _tc_update2(h1, sum2, cnt3, scal, c2uW[0:_H], c2uW[_H:], c2ub,
                      fc1W[0:_H], fc1W[_H:_H + 2], fc1b, fc2W, fc2b)
    return out.reshape((1,))


# speedup vs baseline: 5.8578x; 2.1456x over previous
import functools

import jax
import jax.numpy as jnp
from jax import lax
from jax.experimental import pallas as pl
from jax.experimental.pallas import tpu as pltpu
from jax.experimental.pallas import tpu_sc as plsc

F32 = jnp.float32

_N = 50000
_E = 800000
_H = 64
_NC = 2
_NS = 16
_NW = _NC * _NS
_EPW = _E // _NW
_NPAD = 51200
_RPT = _NPAD // _NS
_PZ = 800
_NPZ = _RPT // _PZ
_QW = 16
_C = 1000
_NCH = _EPW // _C

_BE = 4000
_NBE = _E // _BE
_BN = 2000
_NBN = _N // _BN


def _sc_mesh():
    return plsc.VectorSubcoreMesh(core_axis_name="c", subcore_axis_name="s",
                                  num_cores=_NC, num_subcores=_NS)


_SC_PARAMS = pltpu.CompilerParams(use_tc_tiling_on_sc=False)


def _full(shape):
    return pl.BlockSpec(shape, lambda i: tuple(0 for _ in shape))


def _sc_gather_ab(AB, srcE, dstE):
    @functools.partial(
        pl.kernel,
        out_type=jax.ShapeDtypeStruct((_E, 128), F32),
        mesh=_sc_mesh(),
        compiler_params=_SC_PARAMS,
        scratch_types=[
            pltpu.VMEM((_C,), jnp.int32),
            pltpu.VMEM((_C, 128), F32),
            pltpu.SemaphoreType.DMA,
        ],
    )
    def k(ab_h, src_h, dst_h, gab_h, idx_v, buf_v, sem):
        cid = lax.axis_index("c")
        sid = lax.axis_index("s")
        wid = sid * _NC + cid
        base = wid * _EPW

        def chunk(kk, carry):
            e0 = base + kk * _C
            pltpu.sync_copy(dst_h.at[pl.ds(e0, _C)], idx_v)
            pltpu.async_copy(ab_h.at[idx_v], buf_v, sem).wait()
            pltpu.sync_copy(buf_v.at[:, pl.ds(0, _H)],
                            gab_h.at[pl.ds(e0, _C), pl.ds(0, _H)])
            pltpu.sync_copy(src_h.at[pl.ds(e0, _C)], idx_v)
            pltpu.async_copy(ab_h.at[idx_v], buf_v, sem).wait()
            pltpu.sync_copy(buf_v.at[:, pl.ds(_H, _H)],
                            gab_h.at[pl.ds(e0, _C), pl.ds(_H, _H)])
            return carry

        lax.fori_loop(0, _NCH, chunk, 0)

    return k(AB, srcE, dstE)


def _make_sc_scatter(with_counts):
    out_type = [jax.ShapeDtypeStruct((_NC, _NPAD, 128), F32)]
    scratch = [
        pltpu.VMEM((_C,), jnp.int32),
        pltpu.VMEM((_C, _QW), F32),
        pltpu.VMEM((_PZ, _QW), F32),
        pltpu.VMEM_SHARED((_NPAD, _QW), F32),
    ]
    if with_counts:
        out_type.append(jax.ShapeDtypeStruct((_NC * _NPAD,), F32))
        scratch += [
            pltpu.VMEM((_C,), F32),
            pltpu.VMEM((_RPT,), F32),
            pltpu.VMEM_SHARED((_NPAD,), F32),
        ]

    def body(m_h, dst_h, z_h, zc_h, ones_h, sum_h, cnt_h,
             idx_v, data_v, zb_v, tab_sh, ones_v, cbuf_v, cnt_sh):
        cid = lax.axis_index("c")
        sid = lax.axis_index("s")
        wid = sid * _NC + cid
        base = wid * _EPW
        pltpu.sync_copy(z_h, zb_v)
        if with_counts:
            pltpu.sync_copy(zc_h, cbuf_v)
            pltpu.sync_copy(cbuf_v, cnt_sh.at[pl.ds(sid * _RPT, _RPT)])
            pltpu.sync_copy(ones_h, ones_v)
        for q in range(4):
            for p in range(_NPZ):
                pltpu.sync_copy(
                    zb_v, tab_sh.at[pl.ds(sid * _RPT + p * _PZ, _PZ), :])
            plsc.subcore_barrier()

            def chunk(kk, carry):
                e0 = base + kk * _C
                pltpu.sync_copy(dst_h.at[pl.ds(e0, _C)], idx_v)
                pltpu.sync_copy(m_h.at[pl.ds(e0, _C), pl.ds(_QW * q, _QW)],
                                data_v)
                pltpu.sync_copy(data_v, tab_sh.at[idx_v], add=True)
                if with_counts and q == 0:
                    pltpu.sync_copy(ones_v, cnt_sh.at[idx_v], add=True)
                return carry

            lax.fori_loop(0, _NCH, chunk, 0)
            plsc.subcore_barrier()
            for p in range(_NPZ):
                r0 = sid * _RPT + p * _PZ
                pltpu.sync_copy(tab_sh.at[pl.ds(r0, _PZ), :], zb_v)
                pltpu.sync_copy(zb_v, sum_h.at[cid, pl.ds(r0, _PZ),
                                               pl.ds(_QW * q, _QW)])
            pltpu.sync_copy(z_h, zb_v)
            plsc.subcore_barrier()
        if with_counts:
            pltpu.sync_copy(cnt_sh.at[pl.ds(sid * _RPT, _RPT)], cbuf_v)
            pltpu.sync_copy(cbuf_v,
                            cnt_h.at[pl.ds(cid * _NPAD + sid * _RPT, _RPT)])

    if with_counts:
        def k(m_h, dst_h, z_h, zc_h, ones_h, sum_h, cnt_h,
              idx_v, data_v, zb_v, tab_sh, ones_v, cbuf_v, cnt_sh):
            body(m_h, dst_h, z_h, zc_h, ones_h, sum_h, cnt_h,
                 idx_v, data_v, zb_v, tab_sh, ones_v, cbuf_v, cnt_sh)
    else:
        def k(m_h, dst_h, z_h, sum_h, idx_v, data_v, zb_v, tab_sh):
            body(m_h, dst_h, z_h, None, None, sum_h, None,
                 idx_v, data_v, zb_v, tab_sh, None, None, None)

    return functools.partial(
        pl.kernel,
        out_type=tuple(out_type) if with_counts else out_type[0],
        mesh=_sc_mesh(),
        compiler_params=_SC_PARAMS,
        scratch_types=scratch,
    )(k)


def _sc_scatter_counts(msg, dstE, zq, zcnt, onesC):
    return _make_sc_scatter(True)(msg, dstE, zq, zcnt, onesC)


def _sc_scatter(msg, dstE, zq):
    return _make_sc_scatter(False)(msg, dstE, zq)


def _tc_ab1(x, W1, b1):
    def body(x_r, W1_r, b1_r, ab_r):
        L = x_r[...] * W1_r[0:1, :] + b1_r[...]
        R = x_r[...] * W1_r[1:2, :]
        ab_r[...] = jnp.concatenate([L, R], axis=1)

    return pl.pallas_call(
        body,
        grid=(_NBN,),
        in_specs=[pl.BlockSpec((_BN, 1), lambda i: (i, 0)),
                  _full((3, _H)), _full((_H,))],
        out_specs=pl.BlockSpec((_BN, 128), lambda i: (i, 0)),
        out_shape=jax.ShapeDtypeStruct((_N, 128), F32),
    )(x, W1, b1)


def _tc_msg(GAB, ea, we, W2, b2):
    def body(g_r, ea_r, we_r, W2_r, b2_r, m_r):
        pre = g_r[:, 0:_H] + g_r[:, _H:128] + ea_r[...] * we_r[...]
        m = jnp.maximum(pre, 0.0)
        out = jnp.dot(m, W2_r[...], preferred_element_type=F32) + b2_r[...]
        m_r[:, 0:_H] = out

    return pl.pallas_call(
        body,
        grid=(_NBE,),
        in_specs=[pl.BlockSpec((_BE, 128), lambda i: (i, 0)),
                  pl.BlockSpec((_BE, 1), lambda i: (i, 0)),
                  _full((1, _H)), _full((_H, _H)), _full((_H,))],
        out_specs=pl.BlockSpec((_BE, 128), lambda i: (i, 0)),
        out_shape=jax.ShapeDtypeStruct((_E, 128), F32),
    )(GAB, ea, we, W2, b2)


def _tc_update1(x, sumP, cntP, uWx, uWa, ub, Wd, bm, Ws):
    def body(x_r, S_r, c_r, uWx_r, uWa_r, ub_r, Wd_r, bm_r, Ws_r,
             h1_r, ab_r):
        s = S_r[0, :, 0:_H] + S_r[1, :, 0:_H]
        cnt = jnp.maximum(c_r[0] + c_r[1], 1.0)
        agg = s / cnt
        u = (x_r[...] * uWx_r[...]
             + jnp.dot(agg, uWa_r[...], preferred_element_type=F32) + ub_r[...])
        h = jnp.maximum(u, 0.0)
        h1_r[...] = h
        A = jnp.dot(h, Wd_r[...], preferred_element_type=F32) + bm_r[...]
        B = jnp.dot(h, Ws_r[...], preferred_element_type=F32)
        ab_r[...] = jnp.concatenate([A, B], axis=1)

    return pl.pallas_call(
        body,
        grid=(_NBN,),
        in_specs=[
            pl.BlockSpec((_BN, 1), lambda i: (i, 0)),
            pl.BlockSpec((2, _BN, 128), lambda i: (0, i, 0)),
            pl.BlockSpec((2, _BN, 1), lambda i: (0, i, 0)),
            _full((1, _H)), _full((_H, _H)), _full((_H,)),
            _full((_H, _H)), _full((_H,)), _full((_H, _H)),
        ],
        out_specs=[pl.BlockSpec((_BN, _H), lambda i: (i, 0)),
                   pl.BlockSpec((_BN, 128), lambda i: (i, 0))],
        out_shape=[jax.ShapeDtypeStruct((_N, _H), F32),
                   jax.ShapeDtypeStruct((_N, 128), F32)],
    )(x, sumP, cntP, uWx, uWa, ub, Wd, bm, Ws)


def _tc_update2(h1, sumP, cntP, scal, uWh, uWa, ub, fc1Wp, fc1s, fc1b,
                fc2W, fc2b):
    def body(h1_r, S_r, c_r, scal_r, uWh_r, uWa_r, ub_r, W1p_r, W1s_r, b1_r,
             W2_r, b2_r, out_r, acc):
        i = pl.program_id(0)
        s = S_r[0, :, 0:_H] + S_r[1, :, 0:_H]
        cnt = jnp.maximum(c_r[0] + c_r[1], 1.0)
        agg = s / cnt
        u = (jnp.dot(h1_r[...], uWh_r[...], preferred_element_type=F32)
             + jnp.dot(agg, uWa_r[...], preferred_element_type=F32)
             + ub_r[...])
        h2 = jnp.maximum(u, 0.0)
        ps = jnp.sum(h2, axis=0, keepdims=True)

        @pl.when(i == 0)
        def _init():
            acc[0:1, 0:_H] = ps

        @pl.when(i > 0)
        def _accum():
            acc[0:1, 0:_H] = acc[0:1, 0:_H] + ps

        @pl.when(i == _NBN - 1)
        def _finish():
            pooled = acc[0:1, 0:_H] / jnp.float32(_N)
            u1 = (jnp.dot(pooled, W1p_r[...], preferred_element_type=F32)
                  + scal_r[0] * W1s_r[0:1, :] + scal_r[1] * W1s_r[1:2, :]
                  + b1_r[...])
            r1 = jnp.maximum(u1, 0.0)
            z = jnp.dot(r1, W2_r[...], preferred_element_type=F32) + b2_r[...]
            out_r[...] = z

    return pl.pallas_call(
        body,
        grid=(_NBN,),
        in_specs=[
            pl.BlockSpec((_BN, _H), lambda i: (i, 0)),
            pl.BlockSpec((2, _BN, 128), lambda i: (0, i, 0)),
            pl.BlockSpec((2, _BN, 1), lambda i: (0, i, 0)),
            pl.BlockSpec(memory_space=pltpu.SMEM),
            _full((_H, _H)), _full((_H, _H)), _full((_H,)),
            _full((_H, _H)), _full((2, _H)), _full((_H,)),
            _full((_H, 1)), _full((1,)),
        ],
        out_specs=pl.BlockSpec((1, 1), lambda i: (0, 0)),
        out_shape=jax.ShapeDtypeStruct((1, 1), F32),
        scratch_shapes=[pltpu.VMEM((8, 128), F32)],
    )(h1, sumP, cntP, scal, uWh, uWa, ub, fc1Wp, fc1s, fc1b, fc2W, fc2b)


def kernel(x, edge_index, edge_attr, theta, log_h, batch,
           c1m1W, c1m1b, c1m2W, c1m2b, c1uW, c1ub,
           c2m1W, c2m1b, c2m2W, c2m2b, c2uW, c2ub,
           fc1W, fc1b, fc2W, fc2b):
    src = edge_index[0]
    dst = edge_index[1]
    onesC = jnp.ones((_C,), F32)
    zcnt = jnp.zeros((_RPT,), F32)
    zq = jnp.zeros((_PZ, _QW), F32)

    AB1 = _tc_ab1(x, c1m1W, c1m1b)
    GAB1 = _sc_gather_ab(AB1, src, dst)
    msg1 = _tc_msg(GAB1, edge_attr, c1m1W[2:3], c1m2W, c1m2b)
    sum1, cntP = _sc_scatter_counts(msg1, dst, zq, zcnt, onesC)
    cnt3 = cntP.reshape((_NC, _NPAD, 1))
    h1, AB2 = _tc_update1(x, sum1, cnt3, c1uW[0:1], c1uW[1:], c1ub,
                          c2m1W[0:_H], c2m1b, c2m1W[_H:2 * _H])
    GAB2 = _sc_gather_ab(AB2, src, dst)
    msg2 = _tc_msg(GAB2, edge_attr, c2m1W[2 * _H:2 * _H + 1], c2m2W, c2m2b)
    sum2 = _sc_scatter(msg2, dst, zq)
    scal = jnp.concatenate([theta, log_h]).astype(F32)
    out = _tc_update2(h1, sum2, cnt3, scal, c2uW[0:_H], c2uW[_H:], c2ub,
                      fc1W[0:_H], fc1W[_H:_H + 2], fc1b, fc2W, fc2b)
    return out.reshape((1,))


# speedup vs baseline: 6.3417x; 1.0826x over previous
import functools

import jax
import jax.numpy as jnp
from jax import lax
from jax.experimental import pallas as pl
from jax.experimental.pallas import tpu as pltpu
from jax.experimental.pallas import tpu_sc as plsc

F32 = jnp.float32

_N = 50000
_E = 800000
_H = 64
_NC = 2
_NS = 16
_NW = _NC * _NS
_EPW = _E // _NW
_NPAD = 51200
_RPT = _NPAD // _NS
_PZ = 800
_NPZ = _RPT // _PZ
_QW = 16
_C = 1000
_NCH = _EPW // _C

_BE = 4000
_NBE = _E // _BE
_BN = 2000
_NBN = _N // _BN


def _sc_mesh():
    return plsc.VectorSubcoreMesh(core_axis_name="c", subcore_axis_name="s",
                                  num_cores=_NC, num_subcores=_NS)


_SC_PARAMS = pltpu.CompilerParams(use_tc_tiling_on_sc=False)


def _full(shape):
    return pl.BlockSpec(shape, lambda i: tuple(0 for _ in shape))


def _sc_gather_ab(AB, srcE, dstE):
    @functools.partial(
        pl.kernel,
        out_type=jax.ShapeDtypeStruct((_E, 128), F32),
        mesh=_sc_mesh(),
        compiler_params=_SC_PARAMS,
        scratch_types=[
            pltpu.VMEM((_C,), jnp.int32),
            pltpu.VMEM((_C, 128), F32),
            pltpu.SemaphoreType.DMA,
        ],
    )
    def k(ab_h, src_h, dst_h, gab_h, idx_v, buf_v, sem):
        cid = lax.axis_index("c")
        sid = lax.axis_index("s")
        wid = sid * _NC + cid
        base = wid * _EPW

        def chunk(kk, carry):
            e0 = base + kk * _C
            pltpu.sync_copy(dst_h.at[pl.ds(e0, _C)], idx_v)
            pltpu.async_copy(ab_h.at[idx_v], buf_v, sem).wait()
            pltpu.sync_copy(buf_v.at[:, pl.ds(0, _H)],
                            gab_h.at[pl.ds(e0, _C), pl.ds(0, _H)])
            pltpu.sync_copy(src_h.at[pl.ds(e0, _C)], idx_v)
            pltpu.async_copy(ab_h.at[idx_v], buf_v, sem).wait()
            pltpu.sync_copy(buf_v.at[:, pl.ds(_H, _H)],
                            gab_h.at[pl.ds(e0, _C), pl.ds(_H, _H)])
            return carry

        lax.fori_loop(0, _NCH, chunk, 0)

    return k(AB, srcE, dstE)


def _make_sc_scatter(with_counts):
    out_type = [jax.ShapeDtypeStruct((_NC, _NPAD, 128), F32)]
    scratch = [
        pltpu.VMEM((2, _C), jnp.int32),
        pltpu.VMEM((2, _C, _QW), F32),
        pltpu.VMEM((_PZ, _QW), F32),
        pltpu.VMEM_SHARED((_NPAD, _QW), F32),
        pltpu.SemaphoreType.DMA((2,)),
        pltpu.SemaphoreType.DMA((2,)),
        pltpu.SemaphoreType.DMA((2,)),
        pltpu.SemaphoreType.DMA,
    ]
    if with_counts:
        out_type.append(jax.ShapeDtypeStruct((_NC * _NPAD,), F32))
        scratch += [
            pltpu.VMEM((_C,), F32),
            pltpu.VMEM((_RPT,), F32),
            pltpu.VMEM_SHARED((_NPAD,), F32),
            pltpu.SemaphoreType.DMA((2,)),
        ]

    def body(m_h, dst_h, z_h, zc_h, ones_h, sum_h, cnt_h,
             idx_v, data_v, zb_v, tab_sh, sem_i, sem_d, sem_s, sem_z,
             ones_v, cbuf_v, cnt_sh, sem_c):
        cid = lax.axis_index("c")
        sid = lax.axis_index("s")
        wid = sid * _NC + cid
        base = wid * _EPW
        pltpu.sync_copy(z_h, zb_v)
        if with_counts:
            pltpu.sync_copy(zc_h, cbuf_v)
            pltpu.sync_copy(cbuf_v, cnt_sh.at[pl.ds(sid * _RPT, _RPT)])
            pltpu.sync_copy(ones_h, ones_v)

        for q in range(4):
            do_cnt = with_counts and q == 0

            def issue_loads(cc, s):
                e0 = base + cc * _C
                pltpu.async_copy(dst_h.at[pl.ds(e0, _C)], idx_v.at[s],
                                 sem_i.at[s])
                pltpu.async_copy(m_h.at[pl.ds(e0, _C), pl.ds(_QW * q, _QW)],
                                 data_v.at[s], sem_d.at[s])

            def wait_loads(s):
                pltpu.make_async_copy(dst_h.at[pl.ds(0, _C)], idx_v.at[s],
                                      sem_i.at[s]).wait()
                pltpu.make_async_copy(m_h.at[pl.ds(0, _C), pl.ds(0, _QW)],
                                      data_v.at[s], sem_d.at[s]).wait()

            def wait_scat(s):
                pltpu.make_async_copy(data_v.at[s], tab_sh.at[idx_v.at[s]],
                                      sem_s.at[s]).wait()
                if do_cnt:
                    pltpu.make_async_copy(ones_v, cnt_sh.at[idx_v.at[s]],
                                          sem_c.at[s]).wait()

            for p in range(_NPZ):
                pltpu.async_copy(
                    zb_v, tab_sh.at[pl.ds(sid * _RPT + p * _PZ, _PZ), :],
                    sem_z)
            for p in range(_NPZ):
                pltpu.make_async_copy(
                    zb_v, tab_sh.at[pl.ds(sid * _RPT, _PZ), :], sem_z).wait()
            plsc.subcore_barrier()

            issue_loads(0, 0)

            def step(kk, s, o):
                wait_loads(s)

                @pl.when(kk + 1 < _NCH)
                def _more():
                    @pl.when(kk >= 1)
                    def _drain():
                        wait_scat(o)
                    issue_loads(kk + 1, o)

                pltpu.async_copy(data_v.at[s], tab_sh.at[idx_v.at[s]],
                                 sem_s.at[s], add=True)
                if do_cnt:
                    pltpu.async_copy(ones_v, cnt_sh.at[idx_v.at[s]],
                                     sem_c.at[s], add=True)

            def chunk(kk, carry):
                @pl.when(kk % 2 == 0)
                def _a():
                    step(kk, 0, 1)

                @pl.when(kk % 2 == 1)
                def _b():
                    step(kk, 1, 0)

                return carry

            lax.fori_loop(0, _NCH, chunk, 0)
            wait_scat(0)
            wait_scat(1)
            plsc.subcore_barrier()
            for p in range(_NPZ):
                r0 = sid * _RPT + p * _PZ
                pltpu.sync_copy(tab_sh.at[pl.ds(r0, _PZ), :], zb_v)
                pltpu.sync_copy(zb_v, sum_h.at[cid, pl.ds(r0, _PZ),
                                               pl.ds(_QW * q, _QW)])
            pltpu.sync_copy(z_h, zb_v)
            plsc.subcore_barrier()
        if with_counts:
            pltpu.sync_copy(cnt_sh.at[pl.ds(sid * _RPT, _RPT)], cbuf_v)
            pltpu.sync_copy(cbuf_v,
                            cnt_h.at[pl.ds(cid * _NPAD + sid * _RPT, _RPT)])

    if with_counts:
        def k(m_h, dst_h, z_h, zc_h, ones_h, sum_h, cnt_h,
              idx_v, data_v, zb_v, tab_sh, sem_i, sem_d, sem_s, sem_z,
              ones_v, cbuf_v, cnt_sh, sem_c):
            body(m_h, dst_h, z_h, zc_h, ones_h, sum_h, cnt_h,
                 idx_v, data_v, zb_v, tab_sh, sem_i, sem_d, sem_s, sem_z,
                 ones_v, cbuf_v, cnt_sh, sem_c)
    else:
        def k(m_h, dst_h, z_h, sum_h,
              idx_v, data_v, zb_v, tab_sh, sem_i, sem_d, sem_s, sem_z):
            body(m_h, dst_h, z_h, None, None, sum_h, None,
                 idx_v, data_v, zb_v, tab_sh, sem_i, sem_d, sem_s, sem_z,
                 None, None, None, None)

    return functools.partial(
        pl.kernel,
        out_type=tuple(out_type) if with_counts else out_type[0],
        mesh=_sc_mesh(),
        compiler_params=_SC_PARAMS,
        scratch_types=scratch,
    )(k)


def _sc_scatter_counts(msg, dstE, zq, zcnt, onesC):
    return _make_sc_scatter(True)(msg, dstE, zq, zcnt, onesC)


def _sc_scatter(msg, dstE, zq):
    return _make_sc_scatter(False)(msg, dstE, zq)


def _tc_ab1(x, W1, b1):
    def body(x_r, W1_r, b1_r, ab_r):
        L = x_r[...] * W1_r[0:1, :] + b1_r[...]
        R = x_r[...] * W1_r[1:2, :]
        ab_r[...] = jnp.concatenate([L, R], axis=1)

    return pl.pallas_call(
        body,
        grid=(_NBN,),
        in_specs=[pl.BlockSpec((_BN, 1), lambda i: (i, 0)),
                  _full((3, _H)), _full((_H,))],
        out_specs=pl.BlockSpec((_BN, 128), lambda i: (i, 0)),
        out_shape=jax.ShapeDtypeStruct((_N, 128), F32),
    )(x, W1, b1)


def _tc_msg(GAB, ea, we, W2, b2):
    def body(g_r, ea_r, we_r, W2_r, b2_r, m_r):
        pre = g_r[:, 0:_H] + g_r[:, _H:128] + ea_r[...] * we_r[...]
        m = jnp.maximum(pre, 0.0)
        out = jnp.dot(m, W2_r[...], preferred_element_type=F32) + b2_r[...]
        m_r[:, 0:_H] = out

    return pl.pallas_call(
        body,
        grid=(_NBE,),
        in_specs=[pl.BlockSpec((_BE, 128), lambda i: (i, 0)),
                  pl.BlockSpec((_BE, 1), lambda i: (i, 0)),
                  _full((1, _H)), _full((_H, _H)), _full((_H,))],
        out_specs=pl.BlockSpec((_BE, 128), lambda i: (i, 0)),
        out_shape=jax.ShapeDtypeStruct((_E, 128), F32),
    )(GAB, ea, we, W2, b2)


def _tc_update1(x, sumP, cntP, uWx, uWa, ub, Wd, bm, Ws):
    def body(x_r, S_r, c_r, uWx_r, uWa_r, ub_r, Wd_r, bm_r, Ws_r,
             h1_r, ab_r):
        s = S_r[0, :, 0:_H] + S_r[1, :, 0:_H]
        cnt = jnp.maximum(c_r[0] + c_r[1], 1.0)
        agg = s / cnt
        u = (x_r[...] * uWx_r[...]
             + jnp.dot(agg, uWa_r[...], preferred_element_type=F32) + ub_r[...])
        h = jnp.maximum(u, 0.0)
        h1_r[...] = h
        A = jnp.dot(h, Wd_r[...], preferred_element_type=F32) + bm_r[...]
        B = jnp.dot(h, Ws_r[...], preferred_element_type=F32)
        ab_r[...] = jnp.concatenate([A, B], axis=1)

    return pl.pallas_call(
        body,
        grid=(_NBN,),
        in_specs=[
            pl.BlockSpec((_BN, 1), lambda i: (i, 0)),
            pl.BlockSpec((2, _BN, 128), lambda i: (0, i, 0)),
            pl.BlockSpec((2, _BN, 1), lambda i: (0, i, 0)),
            _full((1, _H)), _full((_H, _H)), _full((_H,)),
            _full((_H, _H)), _full((_H,)), _full((_H, _H)),
        ],
        out_specs=[pl.BlockSpec((_BN, _H), lambda i: (i, 0)),
                   pl.BlockSpec((_BN, 128), lambda i: (i, 0))],
        out_shape=[jax.ShapeDtypeStruct((_N, _H), F32),
                   jax.ShapeDtypeStruct((_N, 128), F32)],
    )(x, sumP, cntP, uWx, uWa, ub, Wd, bm, Ws)


def _tc_update2(h1, sumP, cntP, scal, uWh, uWa, ub, fc1Wp, fc1s, fc1b,
                fc2W, fc2b):
    def body(h1_r, S_r, c_r, scal_r, uWh_r, uWa_r, ub_r, W1p_r, W1s_r, b1_r,
             W2_r, b2_r, out_r, acc):
        i = pl.program_id(0)
        s = S_r[0, :, 0:_H] + S_r[1, :, 0:_H]
        cnt = jnp.maximum(c_r[0] + c_r[1], 1.0)
        agg = s / cnt
        u = (jnp.dot(h1_r[...], uWh_r[...], preferred_element_type=F32)
             + jnp.dot(agg, uWa_r[...], preferred_element_type=F32)
             + ub_r[...])
        h2 = jnp.maximum(u, 0.0)
        ps = jnp.sum(h2, axis=0, keepdims=True)

        @pl.when(i == 0)
        def _init():
            acc[0:1, 0:_H] = ps

        @pl.when(i > 0)
        def _accum():
            acc[0:1, 0:_H] = acc[0:1, 0:_H] + ps

        @pl.when(i == _NBN - 1)
        def _finish():
            pooled = acc[0:1, 0:_H] / jnp.float32(_N)
            u1 = (jnp.dot(pooled, W1p_r[...], preferred_element_type=F32)
                  + scal_r[0] * W1s_r[0:1, :] + scal_r[1] * W1s_r[1:2, :]
                  + b1_r[...])
            r1 = jnp.maximum(u1, 0.0)
            z = jnp.dot(r1, W2_r[...], preferred_element_type=F32) + b2_r[...]
            out_r[...] = z

    return pl.pallas_call(
        body,
        grid=(_NBN,),
        in_specs=[
            pl.BlockSpec((_BN, _H), lambda i: (i, 0)),
            pl.BlockSpec((2, _BN, 128), lambda i: (0, i, 0)),
            pl.BlockSpec((2, _BN, 1), lambda i: (0, i, 0)),
            pl.BlockSpec(memory_space=pltpu.SMEM),
            _full((_H, _H)), _full((_H, _H)), _full((_H,)),
            _full((_H, _H)), _full((2, _H)), _full((_H,)),
            _full((_H, 1)), _full((1,)),
        ],
        out_specs=pl.BlockSpec((1, 1), lambda i: (0, 0)),
        out_shape=jax.ShapeDtypeStruct((1, 1), F32),
        scratch_shapes=[pltpu.VMEM((8, 128), F32)],
    )(h1, sumP, cntP, scal, uWh, uWa, ub, fc1Wp, fc1s, fc1b, fc2W, fc2b)


def kernel(x, edge_index, edge_attr, theta, log_h, batch,
           c1m1W, c1m1b, c1m2W, c1m2b, c1uW, c1ub,
           c2m1W, c2m1b, c2m2W, c2m2b, c2uW, c2ub,
           fc1W, fc1b, fc2W, fc2b):
    src = edge_index[0]
    dst = edge_index[1]
    onesC = jnp.ones((_C,), F32)
    zcnt = jnp.zeros((_RPT,), F32)
    zq = jnp.zeros((_PZ, _QW), F32)

    AB1 = _tc_ab1(x, c1m1W, c1m1b)
    GAB1 = _sc_gather_ab(AB1, src, dst)
    msg1 = _tc_msg(GAB1, edge_attr, c1m1W[2:3], c1m2W, c1m2b)
    sum1, cntP = _sc_scatter_counts(msg1, dst, zq, zcnt, onesC)
    cnt3 = cntP.reshape((_NC, _NPAD, 1))
    h1, AB2 = _tc_update1(x, sum1, cnt3, c1uW[0:1], c1uW[1:], c1ub,
                          c2m1W[0:_H], c2m1b, c2m1W[_H:2 * _H])
    GAB2 = _sc_gather_ab(AB2, src, dst)
    msg2 = _tc_msg(GAB2, edge_attr, c2m1W[2 * _H:2 * _H + 1], c2m2W, c2m2b)
    sum2 = _sc_scatter(msg2, dst, zq)
    scal = jnp.concatenate([theta, log_h]).astype(F32)
    out = _tc_update2(h1, sum2, cnt3, scal, c2uW[0:_H], c2uW[_H:], c2ub,
                      fc1W[0:_H], fc1W[_H:_H + 2], fc1b, fc2W, fc2b)
    return out.reshape((1,))
